# baseline (speedup 1.0000x reference)
import functools

import jax
import jax.numpy as jnp
from jax import lax
from jax.experimental import pallas as pl
from jax.experimental.pallas import tpu as pltpu
from jax.experimental.pallas import tpu_sc as plsc

_VOCAB = 100000
_HIDDEN = 1024
_B = 4
_S = 4096
_EPS = 1e-12
_LANES = 16
_NWORKERS = 32
_S_PER_W = _S // _NWORKERS
_CHUNK = 32
_NCHUNK = _S_PER_W // _CHUNK
_HCHUNKS = _HIDDEN // _LANES


def _lane_allsum(x):
    lanes = lax.iota(jnp.int32, _LANES)
    for k in (1, 2, 4, 8):
        idx = lanes ^ k
        x = x + x.at[idx].get(mode="promise_in_bounds")
    return x


def _rsqrt_newton(w):
    i = lax.bitcast_convert_type(w, jnp.int32)
    i = jnp.int32(0x5F3759DF) - lax.shift_right_logical(i, 1)
    y = lax.bitcast_convert_type(i, jnp.float32)
    for _ in range(3):
        y = y * (1.5 - 0.5 * w * y * y)
    return y


def _emb_body(tok_hbm, pos_hbm, ids_hbm, gamma_hbm, beta_hbm, out_hbm,
              pos_v, tok_v, idx_v, g_v, b_v, sem):
    wid = lax.axis_index("s") * 2 + lax.axis_index("c")
    s0 = wid * _S_PER_W

    pltpu.sync_copy(gamma_hbm, g_v)
    pltpu.sync_copy(beta_hbm, b_v)

    def ln_row(r, _):
        def pass1(j, carry):
            s1, s2 = carry
            x = tok_v[r, pl.ds(j * _LANES, _LANES)] + \
                pos_v[r, pl.ds(j * _LANES, _LANES)]
            return s1 + x, s2 + x * x

        zero = jnp.zeros((_LANES,), jnp.float32)
        s1, s2 = lax.fori_loop(0, _HCHUNKS, pass1, (zero, zero))
        t1 = _lane_allsum(s1)
        t2 = _lane_allsum(s2)
        mean_v = t1 * (1.0 / _HIDDEN)
        var_v = t2 * (1.0 / _HIDDEN) - mean_v * mean_v
        rstd_v = _rsqrt_newton(var_v + _EPS)

        def pass2(j, _):
            sl = pl.ds(j * _LANES, _LANES)
            x = tok_v[r, sl] + pos_v[r, sl]
            tok_v[r, sl] = (x - mean_v) * rstd_v * g_v[sl] + b_v[sl]
            return 0

        lax.fori_loop(0, _HCHUNKS, pass2, 0)
        return 0

    for c in range(_NCHUNK):
        sc0 = s0 + c * _CHUNK
        pltpu.sync_copy(pos_hbm.at[pl.ds(sc0, _CHUNK)], pos_v)
        for b in range(_B):
            row0 = b * _S + sc0
            pltpu.sync_copy(ids_hbm.at[pl.ds(row0, _CHUNK)], idx_v)
            pltpu.async_copy(tok_hbm.at[idx_v], tok_v, sem).wait()
            lax.fori_loop(0, _CHUNK, ln_row, 0)
            pltpu.sync_copy(tok_v, out_hbm.at[pl.ds(row0, _CHUNK)])


_emb_kernel = functools.partial(
    pl.kernel,
    out_type=jax.ShapeDtypeStruct((_B * _S, _HIDDEN), jnp.float32),
    mesh=plsc.VectorSubcoreMesh(core_axis_name="c", subcore_axis_name="s"),
    scratch_types=[
        pltpu.VMEM((_CHUNK, _HIDDEN), jnp.float32),
        pltpu.VMEM((_CHUNK, _HIDDEN), jnp.float32),
        pltpu.VMEM((_CHUNK,), jnp.int32),
        pltpu.VMEM((_HIDDEN,), jnp.float32),
        pltpu.VMEM((_HIDDEN,), jnp.float32),
        pltpu.SemaphoreType.DMA,
    ],
)(_emb_body)


def kernel(input_ids, token_table, pos_table, gamma, beta):
    ids = input_ids.reshape(-1).astype(jnp.int32)
    out = _emb_kernel(token_table, pos_table, ids, gamma, beta)
    return out.reshape(_B, _S, _HIDDEN)


# speedup vs baseline: 2.7661x; 2.7661x over previous
import functools

import jax
import jax.numpy as jnp
from jax import lax
from jax.experimental import pallas as pl
from jax.experimental.pallas import tpu as pltpu
from jax.experimental.pallas import tpu_sc as plsc

_VOCAB = 100000
_HIDDEN = 1024
_B = 4
_S = 4096
_EPS = 1e-12
_NWORKERS = 32
_N_ROWS = _S
_ROWS_PER_W = _N_ROWS // _NWORKERS
_GC = 32
_NSUB = _ROWS_PER_W // _GC

_R_BLK = 256
_S_BLKS = _S // _R_BLK


def _gather_body(tok_hbm, ids_hbm, out_hbm, idx_v, buf0, buf1, g0, g1,
                 s0, s1):
    wid = lax.axis_index("s") * 2 + lax.axis_index("c")
    base = wid * _ROWS_PER_W
    pltpu.sync_copy(ids_hbm.at[pl.ds(base, _ROWS_PER_W)], idx_v)

    bufs = (buf0, buf1)
    gsems = (g0, g1)
    ssems = (s0, s1)
    gathers = [None, None]
    scatters = [None, None]
    for g in range(_NSUB + 1):
        if g < _NSUB:
            if g >= 2:
                scatters[g % 2].wait()
            gathers[g % 2] = pltpu.async_copy(
                tok_hbm.at[idx_v.at[pl.ds(g * _GC, _GC)]],
                bufs[g % 2], gsems[g % 2])
        if g >= 1:
            h = g - 1
            gathers[h % 2].wait()
            scatters[h % 2] = pltpu.async_copy(
                bufs[h % 2], out_hbm.at[pl.ds(base + h * _GC, _GC)],
                ssems[h % 2])
    scatters[(_NSUB - 1) % 2].wait()
    scatters[(_NSUB - 2) % 2].wait()


_sc_gather = functools.partial(
    pl.kernel,
    out_type=jax.ShapeDtypeStruct((_N_ROWS, _HIDDEN), jnp.float32),
    mesh=plsc.VectorSubcoreMesh(core_axis_name="c", subcore_axis_name="s"),
    scratch_types=[
        pltpu.VMEM((_ROWS_PER_W,), jnp.int32),
        pltpu.VMEM((_GC, _HIDDEN), jnp.float32),
        pltpu.VMEM((_GC, _HIDDEN), jnp.float32),
        pltpu.SemaphoreType.DMA,
        pltpu.SemaphoreType.DMA,
        pltpu.SemaphoreType.DMA,
        pltpu.SemaphoreType.DMA,
    ],
)(_gather_body)


def _ln_body(tok_ref, pos_ref, g_ref, b_ref, out_ref):
    x = tok_ref[...] + pos_ref[...]
    mean = jnp.mean(x, axis=-1, keepdims=True)
    xc = x - mean
    var = jnp.mean(xc * xc, axis=-1, keepdims=True)
    out_ref[...] = xc * lax.rsqrt(var + _EPS) * g_ref[...] + b_ref[...]


_tc_ln = pl.pallas_call(
    _ln_body,
    grid=(_S_BLKS,),
    in_specs=[
        pl.BlockSpec((_R_BLK, _HIDDEN), lambda s: (s, 0)),
        pl.BlockSpec((_R_BLK, _HIDDEN), lambda s: (s, 0)),
        pl.BlockSpec((1, _HIDDEN), lambda s: (0, 0)),
        pl.BlockSpec((1, _HIDDEN), lambda s: (0, 0)),
    ],
    out_specs=pl.BlockSpec((_R_BLK, _HIDDEN), lambda s: (s, 0)),
    out_shape=jax.ShapeDtypeStruct((_S, _HIDDEN), jnp.float32),
)


def kernel(input_ids, token_table, pos_table, gamma, beta):
    g2 = gamma.reshape(1, _HIDDEN)
    b2 = beta.reshape(1, _HIDDEN)
    outs = []
    for b in range(_B):
        ids_b = input_ids[b].astype(jnp.int32)
        rows_b = _sc_gather(token_table, ids_b)
        outs.append(_tc_ln(rows_b, pos_table, g2, b2))
    return jnp.stack(outs, axis=0)


# speedup vs baseline: 2.8955x; 1.0468x over previous
import functools

import jax
import jax.numpy as jnp
from jax import lax
from jax.experimental import pallas as pl
from jax.experimental.pallas import tpu as pltpu
from jax.experimental.pallas import tpu_sc as plsc

_VOCAB = 100000
_HIDDEN = 1024
_B = 4
_S = 4096
_EPS = 1e-12
_NWORKERS = 32
_N_ROWS = _S
_ROWS_PER_W = _N_ROWS // _NWORKERS
_GC = 32
_NSUB = _ROWS_PER_W // _GC

_R_BLK = 256
_S_BLKS = _S // _R_BLK


def _gather_body(tok_hbm, ids_hbm, out_hbm, idx_v, buf0, buf1, g0, g1,
                 s0, s1):
    wid = lax.axis_index("s") * 2 + lax.axis_index("c")
    base = wid * _ROWS_PER_W
    pltpu.sync_copy(ids_hbm.at[pl.ds(base, _ROWS_PER_W)], idx_v)

    bufs = (buf0, buf1)
    gsems = (g0, g1)
    ssems = (s0, s1)
    gathers = [None, None]
    scatters = [None, None]
    for g in range(_NSUB + 1):
        if g < _NSUB:
            if g >= 2:
                scatters[g % 2].wait()
            gathers[g % 2] = pltpu.async_copy(
                tok_hbm.at[idx_v.at[pl.ds(g * _GC, _GC)]],
                bufs[g % 2], gsems[g % 2])
        if g >= 1:
            h = g - 1
            gathers[h % 2].wait()
            scatters[h % 2] = pltpu.async_copy(
                bufs[h % 2], out_hbm.at[pl.ds(base + h * _GC, _GC)],
                ssems[h % 2])
    scatters[(_NSUB - 1) % 2].wait()
    scatters[(_NSUB - 2) % 2].wait()


_sc_gather = functools.partial(
    pl.kernel,
    out_type=jax.ShapeDtypeStruct((_N_ROWS, _HIDDEN), jnp.float32),
    mesh=plsc.VectorSubcoreMesh(core_axis_name="c", subcore_axis_name="s"),
    scratch_types=[
        pltpu.VMEM((_ROWS_PER_W,), jnp.int32),
        pltpu.VMEM((_GC, _HIDDEN), jnp.float32),
        pltpu.VMEM((_GC, _HIDDEN), jnp.float32),
        pltpu.SemaphoreType.DMA,
        pltpu.SemaphoreType.DMA,
        pltpu.SemaphoreType.DMA,
        pltpu.SemaphoreType.DMA,
    ],
)(_gather_body)


_N_CHUNK = 4
_S_CHUNK = _S // _N_CHUNK
_SBLK_PER_CHUNK = _S_CHUNK // _R_BLK


def _ln_body(tok_ref, pos_ref, g_ref, b_ref, out_ref):
    x = tok_ref[...] + pos_ref[...]
    mean = jnp.mean(x, axis=-1, keepdims=True)
    xc = x - mean
    var = jnp.mean(xc * xc, axis=-1, keepdims=True)
    out_ref[...] = xc * lax.rsqrt(var + _EPS) * g_ref[...] + b_ref[...]


def _make_tc_ln(chunk):
    pos_base = chunk * _SBLK_PER_CHUNK
    return pl.pallas_call(
        _ln_body,
        grid=(_SBLK_PER_CHUNK, _B),
        in_specs=[
            pl.BlockSpec((_R_BLK, _HIDDEN),
                         lambda s, b: (b * _SBLK_PER_CHUNK + s, 0)),
            pl.BlockSpec((_R_BLK, _HIDDEN),
                         lambda s, b: (pos_base + s, 0)),
            pl.BlockSpec((1, _HIDDEN), lambda s, b: (0, 0)),
            pl.BlockSpec((1, _HIDDEN), lambda s, b: (0, 0)),
        ],
        out_specs=pl.BlockSpec((_R_BLK, _HIDDEN),
                               lambda s, b: (b * _SBLK_PER_CHUNK + s, 0)),
        out_shape=jax.ShapeDtypeStruct((_B * _S_CHUNK, _HIDDEN),
                                       jnp.float32),
    )


_tc_lns = [_make_tc_ln(c) for c in range(_N_CHUNK)]


def kernel(input_ids, token_table, pos_table, gamma, beta):
    g2 = gamma.reshape(1, _HIDDEN)
    b2 = beta.reshape(1, _HIDDEN)
    ids = input_ids.astype(jnp.int32)
    rows = []
    for c in range(_N_CHUNK):
        ids_c = ids[:, c * _S_CHUNK:(c + 1) * _S_CHUNK].reshape(-1)
        rows.append(_sc_gather(token_table, ids_c))
    outs = []
    for c in range(_N_CHUNK):
        o = _tc_lns[c](rows[c], pos_table, g2, b2)
        outs.append(o.reshape(_B, _S_CHUNK, _HIDDEN))
    return jnp.concatenate(outs, axis=1)


# speedup vs baseline: 3.8028x; 1.3133x over previous
import functools

import jax
import jax.numpy as jnp
from jax import lax
from jax.experimental import pallas as pl
from jax.experimental.pallas import tpu as pltpu
from jax.experimental.pallas import tpu_sc as plsc

_VOCAB = 100000
_HIDDEN = 1024
_B = 4
_S = 4096
_EPS = 1e-12

_N_CHUNK = 4
_S_CHUNK = _S // _N_CHUNK
_N_ROWS = _B * _S_CHUNK
_NWORKERS = 32
_ROWS_PER_W = _N_ROWS // _NWORKERS
_W_PER_B = _S_CHUNK // _ROWS_PER_W
_GC = 32
_NSUB = _ROWS_PER_W // _GC

_R_BLK = 256
_SBLK_PER_CHUNK = _S_CHUNK // _R_BLK
_SBLK_TOTAL = _B * _S // _R_BLK


def _make_gather_body(chunk):
    def body(tok_hbm, ids_hbm, out_hbm, idx_v, buf0, buf1, g0, g1, s0, s1):
        wid = lax.axis_index("s") * 2 + lax.axis_index("c")
        b = wid // _W_PER_B
        ids_base = b * _S + chunk * _S_CHUNK + (wid % _W_PER_B) * _ROWS_PER_W
        out_base = wid * _ROWS_PER_W
        pltpu.sync_copy(ids_hbm.at[pl.ds(ids_base, _ROWS_PER_W)], idx_v)

        bufs = (buf0, buf1)
        gsems = (g0, g1)
        ssems = (s0, s1)
        gathers = [None, None]
        scatters = [None, None]
        for g in range(_NSUB + 1):
            if g < _NSUB:
                if g >= 2:
                    scatters[g % 2].wait()
                gathers[g % 2] = pltpu.async_copy(
                    tok_hbm.at[idx_v.at[pl.ds(g * _GC, _GC)]],
                    bufs[g % 2], gsems[g % 2])
            if g >= 1:
                h = g - 1
                gathers[h % 2].wait()
                scatters[h % 2] = pltpu.async_copy(
                    bufs[h % 2], out_hbm.at[pl.ds(out_base + h * _GC, _GC)],
                    ssems[h % 2])
        scatters[(_NSUB - 1) % 2].wait()
        scatters[(_NSUB - 2) % 2].wait()
    return body


def _make_sc_gather(chunk):
    return functools.partial(
        pl.kernel,
        out_type=jax.ShapeDtypeStruct((_N_ROWS, _HIDDEN), jnp.float32),
        mesh=plsc.VectorSubcoreMesh(core_axis_name="c",
                                    subcore_axis_name="s"),
        scratch_types=[
            pltpu.VMEM((_ROWS_PER_W,), jnp.int32),
            pltpu.VMEM((_GC, _HIDDEN), jnp.float32),
            pltpu.VMEM((_GC, _HIDDEN), jnp.float32),
            pltpu.SemaphoreType.DMA,
            pltpu.SemaphoreType.DMA,
            pltpu.SemaphoreType.DMA,
            pltpu.SemaphoreType.DMA,
        ],
    )(_make_gather_body(chunk))


_sc_gathers = [_make_sc_gather(c) for c in range(_N_CHUNK)]


def _ln_math(tok_ref, pos_ref, g_ref, b_ref, out_ref):
    x = tok_ref[...] + pos_ref[...]
    mean = jnp.mean(x, axis=-1, keepdims=True)
    xc = x - mean
    var = jnp.mean(xc * xc, axis=-1, keepdims=True)
    out_ref[...] = xc * lax.rsqrt(var + _EPS) * g_ref[...] + b_ref[...]


def _ln_body_acc(acc_ref, tok_ref, pos_ref, g_ref, b_ref, out_ref):
    del acc_ref
    _ln_math(tok_ref, pos_ref, g_ref, b_ref, out_ref)


def _make_tc_ln(chunk):
    pos0 = chunk * _SBLK_PER_CHUNK
    rows_spec = pl.BlockSpec((_R_BLK, _HIDDEN),
                             lambda s, b: (b * _SBLK_PER_CHUNK + s, 0))
    pos_spec = pl.BlockSpec((_R_BLK, _HIDDEN),
                            lambda s, b: (pos0 + s, 0))
    vec_spec = pl.BlockSpec((1, _HIDDEN), lambda s, b: (0, 0))
    out_spec = pl.BlockSpec(
        (_R_BLK, _HIDDEN),
        lambda s, b: (b * (_S // _R_BLK) + pos0 + s, 0))
    out_shape = jax.ShapeDtypeStruct((_B * _S, _HIDDEN), jnp.float32)
    if chunk == 0:
        return pl.pallas_call(
            _ln_math,
            grid=(_SBLK_PER_CHUNK, _B),
            in_specs=[rows_spec, pos_spec, vec_spec, vec_spec],
            out_specs=out_spec,
            out_shape=out_shape,
        )
    return pl.pallas_call(
        _ln_body_acc,
        grid=(_SBLK_PER_CHUNK, _B),
        in_specs=[pl.BlockSpec(memory_space=pl.ANY),
                  rows_spec, pos_spec, vec_spec, vec_spec],
        out_specs=out_spec,
        out_shape=out_shape,
        input_output_aliases={0: 0},
    )


_tc_lns = [_make_tc_ln(c) for c in range(_N_CHUNK)]


def kernel(input_ids, token_table, pos_table, gamma, beta):
    g2 = gamma.reshape(1, _HIDDEN)
    b2 = beta.reshape(1, _HIDDEN)
    ids = input_ids.reshape(-1).astype(jnp.int32)
    rows = [_sc_gathers[c](token_table, ids) for c in range(_N_CHUNK)]
    acc = _tc_lns[0](rows[0], pos_table, g2, b2)
    for c in range(1, _N_CHUNK):
        acc = _tc_lns[c](acc, rows[c], pos_table, g2, b2)
    return acc.reshape(_B, _S, _HIDDEN)
